# baseline (speedup 1.0000x reference)
import functools

import jax
import jax.numpy as jnp
from jax import lax
from jax.experimental import pallas as pl
from jax.experimental.pallas import tpu as pltpu
from jax.experimental.pallas import tpu_sc as plsc

BATCH = 4096
SEQ_LEN = 200
VOCAB = 6
DIM = 64

N_ROWS = BATCH * SEQ_LEN
NC, NS = 2, 16
NW = NC * NS
ROWS_PER_W = N_ROWS // NW
CHUNK_ROWS = 1024
N_CHUNKS = ROWS_PER_W // CHUNK_ROWS
L = 16


def _mesh():
    return plsc.VectorSubcoreMesh(
        core_axis_name="c", subcore_axis_name="s",
        num_cores=NC, num_subcores=NS)


@functools.partial(
    pl.kernel,
    out_type=jax.ShapeDtypeStruct((N_ROWS * DIM,), jnp.float32),
    mesh=_mesh(),
    compiler_params=pltpu.CompilerParams(
        use_tc_tiling_on_sc=False, needs_layout_passes=False),
    scratch_types=[
        pltpu.VMEM((VOCAB * DIM,), jnp.float32),
        pltpu.VMEM((CHUNK_ROWS,), jnp.int32),
        pltpu.VMEM((CHUNK_ROWS * DIM,), jnp.float32),
        pltpu.SemaphoreType.DMA,
    ],
)
def _expand_kernel(table_hbm, idx_hbm, out_hbm, table_v, idx_v, rows_v, sem):
    wid = lax.axis_index("s") * NC + lax.axis_index("c")
    r_base = wid * ROWS_PER_W

    pltpu.sync_copy(table_hbm, table_v)

    lane_stride = lax.iota(jnp.int32, L) * DIM

    def chunk(t, carry):
        r0 = r_base + t * CHUNK_ROWS
        pltpu.sync_copy(idx_hbm.at[pl.ds(r0, CHUNK_ROWS)], idx_v)

        def expand(g, c2):
            src0 = idx_v[pl.ds(g * L, L)] * DIM
            dst0 = lane_stride + g * (L * DIM)
            for j in range(DIM):
                val = plsc.load_gather(table_v, [src0 + j])
                plsc.store_scatter(rows_v, [dst0 + j], val)
            return c2

        lax.fori_loop(0, CHUNK_ROWS // L, expand, 0)
        pltpu.sync_copy(rows_v, out_hbm.at[pl.ds(r0 * DIM, CHUNK_ROWS * DIM)])
        return carry

    lax.fori_loop(0, N_CHUNKS, chunk, 0)


def kernel(indices, table):
    idx = indices.reshape(N_ROWS).astype(jnp.int32)
    flat = _expand_kernel(table.reshape(VOCAB * DIM), idx)
    return flat.reshape(N_ROWS, DIM)


# speedup vs baseline: 3.7034x; 3.7034x over previous
import functools

import jax
import jax.numpy as jnp
from jax import lax
from jax.experimental import pallas as pl
from jax.experimental.pallas import tpu as pltpu
from jax.experimental.pallas import tpu_sc as plsc

BATCH = 4096
SEQ_LEN = 200
VOCAB = 6
DIM = 64

N_ROWS = BATCH * SEQ_LEN
NC, NS = 2, 16
NW = NC * NS
ROWS_PER_W = N_ROWS // NW
CHUNK_ROWS = 1024
N_CHUNKS = ROWS_PER_W // CHUNK_ROWS
L = 16


def _mesh():
    return plsc.VectorSubcoreMesh(
        core_axis_name="c", subcore_axis_name="s",
        num_cores=NC, num_subcores=NS)


@functools.partial(
    pl.kernel,
    out_type=jax.ShapeDtypeStruct((N_ROWS * DIM,), jnp.float32),
    mesh=_mesh(),
    compiler_params=pltpu.CompilerParams(
        use_tc_tiling_on_sc=False, needs_layout_passes=False),
    scratch_types=[
        pltpu.VMEM((VOCAB * DIM,), jnp.float32),
        pltpu.VMEM((CHUNK_ROWS,), jnp.int32),
        pltpu.VMEM((CHUNK_ROWS * DIM,), jnp.float32),
        pltpu.SemaphoreType.DMA,
    ],
)
def _expand_kernel(table_hbm, idx_hbm, out_hbm, table_v, idx_v, rows_v, sem):
    wid = lax.axis_index("s") * NC + lax.axis_index("c")
    r_base = wid * ROWS_PER_W

    pltpu.sync_copy(table_hbm, table_v)

    def chunk(t, carry):
        r0 = r_base + t * CHUNK_ROWS
        pltpu.sync_copy(idx_hbm.at[pl.ds(r0, CHUNK_ROWS)], idx_v)

        @plsc.parallel_loop(0, CHUNK_ROWS // L, unroll=2)
        def expand(g):
            offs = idx_v[pl.ds(g * L, L)] * DIM
            d0 = g * (L * DIM)
            for k in range(L):
                o = offs[k]
                d = d0 + k * DIM
                for c in range(DIM // L):
                    rows_v[pl.ds(d + c * L, L)] = table_v[pl.ds(o + c * L, L)]

        pltpu.sync_copy(rows_v, out_hbm.at[pl.ds(r0 * DIM, CHUNK_ROWS * DIM)])
        return carry

    lax.fori_loop(0, N_CHUNKS, chunk, 0)


def kernel(indices, table):
    idx = indices.reshape(N_ROWS).astype(jnp.int32)
    flat = _expand_kernel(table.reshape(VOCAB * DIM), idx)
    return flat.reshape(N_ROWS, DIM)


# speedup vs baseline: 4.0000x; 1.0801x over previous
import functools

import jax
import jax.numpy as jnp
from jax import lax
from jax.experimental import pallas as pl
from jax.experimental.pallas import tpu as pltpu
from jax.experimental.pallas import tpu_sc as plsc

BATCH = 4096
SEQ_LEN = 200
VOCAB = 6
DIM = 64

N_ROWS = BATCH * SEQ_LEN
NC, NS = 2, 16
NW = NC * NS
ROWS_PER_W = N_ROWS // NW
CHUNK_ROWS = 512
N_CHUNKS = ROWS_PER_W // CHUNK_ROWS
L = 16


def _mesh():
    return plsc.VectorSubcoreMesh(
        core_axis_name="c", subcore_axis_name="s",
        num_cores=NC, num_subcores=NS)


@functools.partial(
    pl.kernel,
    out_type=jax.ShapeDtypeStruct((N_ROWS * DIM,), jnp.float32),
    mesh=_mesh(),
    compiler_params=pltpu.CompilerParams(
        use_tc_tiling_on_sc=False, needs_layout_passes=False),
    scratch_types=[
        pltpu.VMEM((VOCAB * DIM,), jnp.float32),
        pltpu.VMEM((2, CHUNK_ROWS), jnp.int32),
        pltpu.VMEM((2, CHUNK_ROWS * DIM), jnp.float32),
        pltpu.SemaphoreType.DMA,
        pltpu.SemaphoreType.DMA,
        pltpu.SemaphoreType.DMA,
        pltpu.SemaphoreType.DMA,
    ],
)
def _expand_kernel(table_hbm, idx_hbm, out_hbm, table_v, idx_v, rows_v,
                   sem_i0, sem_i1, sem_o0, sem_o1):
    wid = lax.axis_index("s") * NC + lax.axis_index("c")
    r_base = wid * ROWS_PER_W
    sem_i = (sem_i0, sem_i1)
    sem_o = (sem_o0, sem_o1)

    pltpu.sync_copy(table_hbm, table_v)

    for b in range(2):
        pltpu.async_copy(
            idx_hbm.at[pl.ds(r_base + b * CHUNK_ROWS, CHUNK_ROWS)],
            idx_v.at[b], sem_i[b])

    @pl.loop(0, N_CHUNKS, step=2)
    def chunk_pair(t):
        for b in range(2):
            k = t + b
            r0 = r_base + k * CHUNK_ROWS
            pltpu.make_async_copy(
                idx_hbm.at[pl.ds(r0, CHUNK_ROWS)], idx_v.at[b],
                sem_i[b]).wait()

            @pl.when(t >= 2)
            def _wait_out():
                pltpu.make_async_copy(
                    rows_v.at[b],
                    out_hbm.at[pl.ds(r0 * DIM, CHUNK_ROWS * DIM)],
                    sem_o[b]).wait()

            @plsc.parallel_loop(0, CHUNK_ROWS // L, unroll=2)
            def expand(g):
                offs = idx_v[b, pl.ds(g * L, L)] * DIM
                d0 = g * (L * DIM)
                for kk in range(L):
                    o = offs[kk]
                    d = d0 + kk * DIM
                    for c in range(DIM // L):
                        rows_v[b, pl.ds(d + c * L, L)] = (
                            table_v[pl.ds(o + c * L, L)])

            pltpu.async_copy(
                rows_v.at[b],
                out_hbm.at[pl.ds(r0 * DIM, CHUNK_ROWS * DIM)], sem_o[b])

            k_next = lax.rem(k + 2, N_CHUNKS)
            pltpu.async_copy(
                idx_hbm.at[pl.ds(r_base + k_next * CHUNK_ROWS, CHUNK_ROWS)],
                idx_v.at[b], sem_i[b])

    for b in range(2):
        pltpu.make_async_copy(
            idx_hbm.at[pl.ds(r_base, CHUNK_ROWS)], idx_v.at[b],
            sem_i[b]).wait()
        pltpu.make_async_copy(
            rows_v.at[b],
            out_hbm.at[pl.ds(r_base * DIM, CHUNK_ROWS * DIM)],
            sem_o[b]).wait()


def kernel(indices, table):
    idx = indices.reshape(N_ROWS).astype(jnp.int32)
    flat = _expand_kernel(table.reshape(VOCAB * DIM), idx)
    return flat.reshape(N_ROWS, DIM)


# speedup vs baseline: 5.3269x; 1.3317x over previous
import functools

import jax
import jax.numpy as jnp
from jax import lax
from jax.experimental import pallas as pl
from jax.experimental.pallas import tpu as pltpu
from jax.experimental.pallas import tpu_sc as plsc

BATCH = 4096
SEQ_LEN = 200
VOCAB = 6
DIM = 64

N_ROWS = BATCH * SEQ_LEN
NC, NS = 2, 16
NW = NC * NS
ROWS_PER_W = N_ROWS // NW
CHUNK_ROWS = 256
N_CHUNKS = ROWS_PER_W // CHUNK_ROWS
L = 16


def _mesh():
    return plsc.VectorSubcoreMesh(
        core_axis_name="c", subcore_axis_name="s",
        num_cores=NC, num_subcores=NS)


@functools.partial(
    pl.kernel,
    out_type=jax.ShapeDtypeStruct((N_ROWS, DIM), jnp.float32),
    mesh=_mesh(),
    compiler_params=pltpu.CompilerParams(use_tc_tiling_on_sc=True),
    scratch_types=[
        pltpu.VMEM((VOCAB * DIM,), jnp.float32),
        pltpu.VMEM((ROWS_PER_W,), jnp.int32),
        pltpu.VMEM((2, CHUNK_ROWS, DIM), jnp.float32),
        pltpu.SemaphoreType.DMA,
        pltpu.SemaphoreType.DMA,
    ],
)
def _expand_kernel(table_hbm, idx_hbm, out_hbm, table_v, idx_v, rows_v,
                   sem_o0, sem_o1):
    wid = lax.axis_index("s") * NC + lax.axis_index("c")
    r_base = wid * ROWS_PER_W
    sem_o = (sem_o0, sem_o1)

    pltpu.sync_copy(table_hbm, table_v)
    pltpu.sync_copy(idx_hbm.at[pl.ds(r_base, ROWS_PER_W)], idx_v)

    @pl.loop(0, N_CHUNKS, step=2)
    def chunk_pair(t):
        for b in range(2):
            k = t + b
            r0 = r_base + k * CHUNK_ROWS

            @pl.when(t >= 2)
            def _wait_out():
                pltpu.make_async_copy(
                    rows_v.at[b],
                    out_hbm.at[pl.ds(r0, CHUNK_ROWS)],
                    sem_o[b]).wait()

            @plsc.parallel_loop(0, CHUNK_ROWS // L, unroll=2)
            def expand(g):
                offs = idx_v[pl.ds(k * CHUNK_ROWS + g * L, L)] * DIM
                for kk in range(L):
                    o = offs[kk]
                    r = g * L + kk
                    for c in range(DIM // L):
                        rows_v[b, r, pl.ds(c * L, L)] = (
                            table_v[pl.ds(o + c * L, L)])

            pltpu.async_copy(
                rows_v.at[b],
                out_hbm.at[pl.ds(r0, CHUNK_ROWS)], sem_o[b])

    for b in range(2):
        pltpu.make_async_copy(
            rows_v.at[b],
            out_hbm.at[pl.ds(r_base, CHUNK_ROWS)],
            sem_o[b]).wait()


def kernel(indices, table):
    idx = indices.reshape(N_ROWS).astype(jnp.int32)
    return _expand_kernel(table.reshape(VOCAB * DIM), idx)
